# baseline (speedup 1.0000x reference)
import jax
import jax.numpy as jnp
from jax.experimental import pallas as pl
from jax.experimental.pallas import tpu as pltpu

_CHUNK = 4096


def _linear_kernel(xa_ref, xb_ref, w_ref, b_ref, o_ref):
    w = w_ref[...].astype(jnp.bfloat16)
    bb = b_ref[...]
    o_ref[0, :, :_CHUNK] = jnp.dot(w, xa_ref[0].astype(jnp.bfloat16),
                                   preferred_element_type=jnp.float32) + bb
    o_ref[0, :, _CHUNK:] = jnp.dot(w, xb_ref[0].astype(jnp.bfloat16),
                                   preferred_element_type=jnp.float32) + bb


def kernel(feat_map0, feat_map1, feat_map2, W, b):
    B, C, H, Wd = feat_map0.shape
    P = H * Wd
    x = feat_map0.reshape(B, C, P)
    b2 = b.reshape(C, 1)
    out = pl.pallas_call(
        _linear_kernel,
        grid=(B, P // (2 * _CHUNK)),
        in_specs=[
            pl.BlockSpec((1, C, _CHUNK), lambda i, j: (i, 0, 2 * j)),
            pl.BlockSpec((1, C, _CHUNK), lambda i, j: (i, 0, 2 * j + 1)),
            pl.BlockSpec((C, C), lambda i, j: (0, 0)),
            pl.BlockSpec((C, 1), lambda i, j: (0, 0)),
        ],
        out_specs=pl.BlockSpec((1, C, 2 * _CHUNK), lambda i, j: (i, 0, j)),
        out_shape=jax.ShapeDtypeStruct((B, C, P), jnp.float32),
        compiler_params=pltpu.CompilerParams(
            dimension_semantics=("parallel", "parallel")),
    )(x, x, W, b2)
    return (out.reshape(B, C, H, Wd), feat_map1, feat_map2)


# speedup vs baseline: 1.0093x; 1.0093x over previous
import jax
import jax.numpy as jnp
from jax.experimental import pallas as pl
from jax.experimental.pallas import tpu as pltpu

_CHUNK = 16384


def _linear_kernel(x_ref, w_ref, b_ref, o_ref):
    x = x_ref[0].astype(jnp.bfloat16)
    w = w_ref[...].astype(jnp.bfloat16)
    o_ref[0] = jnp.dot(w, x, preferred_element_type=jnp.float32) + b_ref[...]


def kernel(feat_map0, feat_map1, feat_map2, W, b):
    B, C, H, Wd = feat_map0.shape
    P = H * Wd
    x = feat_map0.reshape(B, C, P)
    b2 = b.reshape(C, 1)
    out = pl.pallas_call(
        _linear_kernel,
        grid=(B, P // _CHUNK),
        in_specs=[
            pl.BlockSpec((1, C, _CHUNK), lambda i, j: (i, 0, j)),
            pl.BlockSpec((C, C), lambda i, j: (0, 0)),
            pl.BlockSpec((C, 1), lambda i, j: (0, 0)),
        ],
        out_specs=pl.BlockSpec((1, C, _CHUNK), lambda i, j: (i, 0, j)),
        out_shape=jax.ShapeDtypeStruct((B, C, P), jnp.float32),
        compiler_params=pltpu.CompilerParams(
            dimension_semantics=("parallel", "parallel")),
    )(x, W, b2)
    return (out.reshape(B, C, H, Wd), feat_map1, feat_map2)
